# baseline (speedup 1.0000x reference)
import functools

import jax
import jax.numpy as jnp
from jax import lax
from jax.experimental import pallas as pl
from jax.experimental.pallas import tpu as pltpu
from jax.experimental.pallas import tpu_sc as plsc

B = 16384
F_DENSE = 13
F = 26
V = 100000
D = 16
NC = 2
NS = 16
HB = B // NC
G = HB // D


def _bi_sc_kernel(idx_hbm, tt_hbm, s1_hbm, s2_hbm,
                  row_v, idx_v, s1_v, s2_v, sem):
    h = lax.axis_index("c")
    c = lax.axis_index("s")
    hbase = h * HB

    for f in range(F):
        pltpu.sync_copy(idx_hbm.at[f, pl.ds(hbase, HB)], idx_v)
        pltpu.sync_copy(tt_hbm.at[f, c, :], row_v)

        if f == 0:
            def body0(g, _):
                sl = pl.ds(g * D, D)
                x = plsc.load_gather(row_v, [idx_v[sl]])
                s1_v[sl] = x
                s2_v[sl] = x * x
                return 0
            lax.fori_loop(0, G, body0, 0)
        else:
            def body(g, _):
                sl = pl.ds(g * D, D)
                x = plsc.load_gather(row_v, [idx_v[sl]])
                s1_v[sl] = s1_v[sl] + x
                s2_v[sl] = s2_v[sl] + x * x
                return 0
            lax.fori_loop(0, G, body, 0)

    pltpu.sync_copy(s1_v, s1_hbm.at[c, pl.ds(hbase, HB)])
    pltpu.sync_copy(s2_v, s2_hbm.at[c, pl.ds(hbase, HB)])


@functools.cache
def _bi_call():
    return pl.kernel(
        _bi_sc_kernel,
        mesh=plsc.VectorSubcoreMesh(
            core_axis_name="c", subcore_axis_name="s", num_cores=NC),
        compiler_params=pltpu.CompilerParams(
            use_tc_tiling_on_sc=True, needs_layout_passes=False),
        out_type=[
            jax.ShapeDtypeStruct((D, B), jnp.float32),
            jax.ShapeDtypeStruct((D, B), jnp.float32),
        ],
        scratch_types=[
            pltpu.VMEM((V,), jnp.float32),
            pltpu.VMEM((HB,), jnp.int32),
            pltpu.VMEM((HB,), jnp.float32),
            pltpu.VMEM((HB,), jnp.float32),
            pltpu.SemaphoreType.DMA,
        ],
    )


def _stats_body(d_ref, s1_ref, s2_ref, od_ref, ob_ref):
    x = d_ref[...]
    od_ref[...] = jnp.concatenate(
        [jnp.sum(x, axis=0, keepdims=True),
         jnp.sum(x * x, axis=0, keepdims=True)], axis=0)
    bi = 0.5 * (s1_ref[...] * s1_ref[...] - s2_ref[...])
    ob_ref[...] = jnp.concatenate(
        [jnp.sum(bi, axis=1, keepdims=True),
         jnp.sum(bi * bi, axis=1, keepdims=True)], axis=1)


def _mlp_body(dense_ref, s1_ref, s2_ref, dstat_ref, bstat_ref,
              gd_ref, gb_ref, bd_ref, bb_ref,
              w1d_ref, w1b_ref, b1_ref, w2_ref, b2_ref, w3_ref, b3_ref,
              wo_ref, bo_ref, o_ref):
    hp = jax.lax.Precision.HIGHEST
    inv_b = 1.0 / B
    dstat = dstat_ref[...]
    bstat = bstat_ref[...]
    md = dstat[0:1] * inv_b
    vd = dstat[1:2] * inv_b - md * md
    mb = bstat[:, 0:1] * inv_b
    vb = bstat[:, 1:2] * inv_b - mb * mb
    rd = lax.rsqrt(vd + 1e-3) * gd_ref[...]
    rb = lax.rsqrt(vb + 1e-3) * gb_ref[...].reshape(D, 1)
    xd = (dense_ref[...] - md) * rd + bd_ref[...]
    bi_c = 0.5 * (s1_ref[...] * s1_ref[...] - s2_ref[...])
    xb_c = (bi_c - mb) * rb + bb_ref[...].reshape(D, 1)
    h = jnp.dot(xd, w1d_ref[...], precision=hp) \
        + lax.dot_general(xb_c, w1b_ref[...], (((0,), (0,)), ((), ())),
                          precision=hp) + b1_ref[...]
    h = jnp.maximum(h, 0.0)
    h = jnp.maximum(jnp.dot(h, w2_ref[...], precision=hp) + b2_ref[...], 0.0)
    h = jnp.maximum(jnp.dot(h, w3_ref[...], precision=hp) + b3_ref[...], 0.0)
    o_ref[...] = jax.nn.sigmoid(jnp.dot(h, wo_ref[...], precision=hp)
                                + bo_ref[...])


def kernel(dense_inputs, sparse_inputs, tables, gamma, beta,
           W1, b1, W2, b2, W3, b3, Wout, bout):
    s1, s2 = _bi_call()(sparse_inputs.T, tables.transpose(0, 2, 1))

    dstats, bstats = pl.pallas_call(
        _stats_body,
        out_shape=[jax.ShapeDtypeStruct((2, F_DENSE), jnp.float32),
                   jax.ShapeDtypeStruct((D, 2), jnp.float32)],
    )(dense_inputs, s1, s2)

    BLK = 2048
    grid = B // BLK
    full = lambda shape: pl.BlockSpec(shape, lambda i: tuple(0 for _ in shape))

    out = pl.pallas_call(
        _mlp_body,
        grid=(grid,),
        in_specs=[
            pl.BlockSpec((BLK, F_DENSE), lambda i: (i, 0)),
            pl.BlockSpec((D, BLK), lambda i: (0, i)),
            pl.BlockSpec((D, BLK), lambda i: (0, i)),
            full((2, F_DENSE)),
            full((D, 2)),
            full((1, F_DENSE)),
            full((1, D)),
            full((1, F_DENSE)),
            full((1, D)),
            full((F_DENSE, 256)),
            full((D, 256)),
            full((1, 256)),
            full((256, 128)),
            full((1, 128)),
            full((128, 64)),
            full((1, 64)),
            full((64, 1)),
            full((1, 1)),
        ],
        out_specs=pl.BlockSpec((BLK, 1), lambda i: (i, 0)),
        out_shape=jax.ShapeDtypeStruct((B, 1), jnp.float32),
    )(
        dense_inputs, s1, s2, dstats, bstats,
        gamma[:F_DENSE].reshape(1, F_DENSE), gamma[F_DENSE:].reshape(1, D),
        beta[:F_DENSE].reshape(1, F_DENSE), beta[F_DENSE:].reshape(1, D),
        W1[:F_DENSE], W1[F_DENSE:], b1.reshape(1, 256),
        W2, b2.reshape(1, 128),
        W3, b3.reshape(1, 64),
        Wout, bout.reshape(1, 1),
    )
    return out


# speedup vs baseline: 1.0151x; 1.0151x over previous
import functools

import jax
import jax.numpy as jnp
from jax import lax
from jax.experimental import pallas as pl
from jax.experimental.pallas import tpu as pltpu
from jax.experimental.pallas import tpu_sc as plsc

B = 16384
F_DENSE = 13
F = 26
V = 100000
D = 16
NC = 2
NS = 16
HB = B // NC
G = HB // D


def _bi_sc_kernel(idx_hbm, tt_hbm, s1_hbm, s2_hbm,
                  row_v, idx_v, s1_v, s2_v, sem):
    h = lax.axis_index("c")
    c = lax.axis_index("s")
    hbase = h * HB

    for f in range(F):
        cp_row = pltpu.async_copy(tt_hbm.at[f, c, :], row_v, sem)
        pltpu.sync_copy(idx_hbm.at[f, pl.ds(hbase, HB)], idx_v)
        cp_row.wait()

        if f == 0:
            def body0(g, _):
                sl = pl.ds(g * D, D)
                x = plsc.load_gather(row_v, [idx_v[sl]])
                s1_v[sl] = x
                s2_v[sl] = x * x
                return 0
            lax.fori_loop(0, G, body0, 0)
        else:
            def body(g, _):
                sl = pl.ds(g * D, D)
                x = plsc.load_gather(row_v, [idx_v[sl]])
                s1_v[sl] = s1_v[sl] + x
                s2_v[sl] = s2_v[sl] + x * x
                return 0
            lax.fori_loop(0, G, body, 0)

    pltpu.sync_copy(s1_v, s1_hbm.at[c, pl.ds(hbase, HB)])
    pltpu.sync_copy(s2_v, s2_hbm.at[c, pl.ds(hbase, HB)])


@functools.cache
def _bi_call():
    return pl.kernel(
        _bi_sc_kernel,
        mesh=plsc.VectorSubcoreMesh(
            core_axis_name="c", subcore_axis_name="s", num_cores=NC),
        compiler_params=pltpu.CompilerParams(
            use_tc_tiling_on_sc=True, needs_layout_passes=False),
        out_type=[
            jax.ShapeDtypeStruct((D, B), jnp.float32),
            jax.ShapeDtypeStruct((D, B), jnp.float32),
        ],
        scratch_types=[
            pltpu.VMEM((V,), jnp.float32),
            pltpu.VMEM((HB,), jnp.int32),
            pltpu.VMEM((HB,), jnp.float32),
            pltpu.VMEM((HB,), jnp.float32),
            pltpu.SemaphoreType.DMA,
        ],
    )


def _stats_body(d_ref, s1_ref, s2_ref, od_ref, ob_ref):
    x = d_ref[...]
    od_ref[...] = jnp.concatenate(
        [jnp.sum(x, axis=0, keepdims=True),
         jnp.sum(x * x, axis=0, keepdims=True)], axis=0)
    bi = 0.5 * (s1_ref[...] * s1_ref[...] - s2_ref[...])
    ob_ref[...] = jnp.concatenate(
        [jnp.sum(bi, axis=1, keepdims=True),
         jnp.sum(bi * bi, axis=1, keepdims=True)], axis=1)


def _mlp_body(dense_ref, s1_ref, s2_ref, dstat_ref, bstat_ref,
              gd_ref, gb_ref, bd_ref, bb_ref,
              w1d_ref, w1b_ref, b1_ref, w2_ref, b2_ref, w3_ref, b3_ref,
              wo_ref, bo_ref, o_ref):
    hp = jax.lax.Precision.HIGHEST
    inv_b = 1.0 / B
    dstat = dstat_ref[...]
    bstat = bstat_ref[...]
    md = dstat[0:1] * inv_b
    vd = dstat[1:2] * inv_b - md * md
    mb = bstat[:, 0:1] * inv_b
    vb = bstat[:, 1:2] * inv_b - mb * mb
    rd = lax.rsqrt(vd + 1e-3) * gd_ref[...]
    rb = lax.rsqrt(vb + 1e-3) * gb_ref[...].reshape(D, 1)
    xd = (dense_ref[...] - md) * rd + bd_ref[...]
    bi_c = 0.5 * (s1_ref[...] * s1_ref[...] - s2_ref[...])
    xb_c = (bi_c - mb) * rb + bb_ref[...].reshape(D, 1)
    h = jnp.dot(xd, w1d_ref[...], precision=hp) \
        + lax.dot_general(xb_c, w1b_ref[...], (((0,), (0,)), ((), ())),
                          precision=hp) + b1_ref[...]
    h = jnp.maximum(h, 0.0)
    h = jnp.maximum(jnp.dot(h, w2_ref[...], precision=hp) + b2_ref[...], 0.0)
    h = jnp.maximum(jnp.dot(h, w3_ref[...], precision=hp) + b3_ref[...], 0.0)
    o_ref[...] = jax.nn.sigmoid(jnp.dot(h, wo_ref[...], precision=hp)
                                + bo_ref[...])


def kernel(dense_inputs, sparse_inputs, tables, gamma, beta,
           W1, b1, W2, b2, W3, b3, Wout, bout):
    s1, s2 = _bi_call()(sparse_inputs.T, tables.transpose(0, 2, 1))

    dstats, bstats = pl.pallas_call(
        _stats_body,
        out_shape=[jax.ShapeDtypeStruct((2, F_DENSE), jnp.float32),
                   jax.ShapeDtypeStruct((D, 2), jnp.float32)],
    )(dense_inputs, s1, s2)

    BLK = 2048
    grid = B // BLK
    full = lambda shape: pl.BlockSpec(shape, lambda i: tuple(0 for _ in shape))

    out = pl.pallas_call(
        _mlp_body,
        grid=(grid,),
        in_specs=[
            pl.BlockSpec((BLK, F_DENSE), lambda i: (i, 0)),
            pl.BlockSpec((D, BLK), lambda i: (0, i)),
            pl.BlockSpec((D, BLK), lambda i: (0, i)),
            full((2, F_DENSE)),
            full((D, 2)),
            full((1, F_DENSE)),
            full((1, D)),
            full((1, F_DENSE)),
            full((1, D)),
            full((F_DENSE, 256)),
            full((D, 256)),
            full((1, 256)),
            full((256, 128)),
            full((1, 128)),
            full((128, 64)),
            full((1, 64)),
            full((64, 1)),
            full((1, 1)),
        ],
        out_specs=pl.BlockSpec((BLK, 1), lambda i: (i, 0)),
        out_shape=jax.ShapeDtypeStruct((B, 1), jnp.float32),
    )(
        dense_inputs, s1, s2, dstats, bstats,
        gamma[:F_DENSE].reshape(1, F_DENSE), gamma[F_DENSE:].reshape(1, D),
        beta[:F_DENSE].reshape(1, F_DENSE), beta[F_DENSE:].reshape(1, D),
        W1[:F_DENSE], W1[F_DENSE:], b1.reshape(1, 256),
        W2, b2.reshape(1, 128),
        W3, b3.reshape(1, 64),
        Wout, bout.reshape(1, 1),
    )
    return out
